# baseline (speedup 1.0000x reference)
import functools

import jax
import jax.numpy as jnp
from jax import lax
from jax.experimental import pallas as pl
from jax.experimental.pallas import tpu as pltpu
from jax.experimental.pallas import tpu_sc as plsc

DIM = 32
NC, NS = 2, 16
NW = NC * NS

BB = 8


@functools.lru_cache(maxsize=None)
def _make_gather(batch, hist):
    rows_per_chunk = BB * hist
    n_chunks = batch // (BB * NW)
    assert batch % (BB * NW) == 0 and n_chunks % 2 == 0
    mesh = plsc.VectorSubcoreMesh(core_axis_name="c", subcore_axis_name="s")

    @functools.partial(
        pl.kernel,
        mesh=mesh,
        out_type=jax.ShapeDtypeStruct((batch * hist, DIM), jnp.float32),
        scratch_types=[
            pltpu.VMEM((rows_per_chunk,), jnp.int32),
            pltpu.VMEM((rows_per_chunk,), jnp.int32),
            pltpu.VMEM((BB * hist, DIM), jnp.float32),
            pltpu.VMEM((BB * hist, DIM), jnp.float32),
            pltpu.SemaphoreType.DMA,
            pltpu.SemaphoreType.DMA,
            pltpu.SemaphoreType.DMA,
            pltpu.SemaphoreType.DMA,
            pltpu.SemaphoreType.DMA,
            pltpu.SemaphoreType.DMA,
        ],
        compiler_params=pltpu.CompilerParams(use_tc_tiling_on_sc=False),
    )
    def gather_kernel(table_hbm, idx_hbm, out_hbm,
                      idx0, idx1, rows0, rows1,
                      si0, si1, sg0, sg1, so0, so1):
        wid = lax.axis_index("s") * NC + lax.axis_index("c")
        base = wid * n_chunks
        idx_v = (idx0, idx1)
        rows_v = (rows0, rows1)
        si = (si0, si1)
        sg = (sg0, sg1)
        so = (so0, so1)

        pltpu.async_copy(idx_hbm.at[base], idx0, si0)
        pltpu.async_copy(idx_hbm.at[base + 1], idx1, si1)

        def step(par, g):
            c = base + g
            r0 = c * BB * hist
            @pl.when(g >= 2)
            def _():
                pltpu.make_async_copy(
                    rows_v[par], out_hbm.at[pl.ds(r0, BB * hist)],
                    so[par]).wait()
            pltpu.make_async_copy(idx_hbm.at[c], idx_v[par], si[par]).wait()
            copies = [
                pltpu.async_copy(
                    table_hbm.at[idx_v[par].at[pl.ds(j * 2 * hist, 2 * hist)]],
                    rows_v[par].at[pl.ds(j * 2 * hist, 2 * hist)], sg[par])
                for j in range(BB // 2)
            ]
            for cp in copies:
                cp.wait()
            pltpu.async_copy(rows_v[par], out_hbm.at[pl.ds(r0, BB * hist)],
                             so[par])
            @pl.when(g + 2 < n_chunks)
            def _():
                pltpu.async_copy(idx_hbm.at[c + 2], idx_v[par], si[par])

        def body(i, carry):
            step(0, 2 * i)
            step(1, 2 * i + 1)
            return carry

        lax.fori_loop(0, n_chunks // 2, body, 0)

        pltpu.make_async_copy(rows0, out_hbm.at[pl.ds(0, BB * hist)],
                              so0).wait()
        pltpu.make_async_copy(rows1, out_hbm.at[pl.ds(0, BB * hist)],
                              so1).wait()

    return gather_kernel


def kernel(residue_type, weight):
    b, h = residue_type.shape
    idx = residue_type.astype(jnp.int32)
    h2 = h // 2
    parts = []
    for lo in (0, h2):
        part = idx[:, lo:lo + h2].reshape(-1, BB * h2)
        parts.append(_make_gather(b, h2)(weight, part).reshape(b, h2, DIM))
    return jnp.concatenate(parts, axis=1)


# speedup vs baseline: 4.7181x; 4.7181x over previous
import functools

import jax
import jax.numpy as jnp
from jax import lax
from jax.experimental import pallas as pl
from jax.experimental.pallas import tpu as pltpu
from jax.experimental.pallas import tpu_sc as plsc

DIM = 32
NC, NS = 2, 16
NW = NC * NS

BB = 8


@functools.lru_cache(maxsize=None)
def _make_gather(batch, hist):
    rows_per_chunk = BB * hist
    n_chunks = batch // (BB * NW)
    assert batch % (BB * NW) == 0 and n_chunks % 2 == 0
    mesh = plsc.VectorSubcoreMesh(core_axis_name="c", subcore_axis_name="s")

    @functools.partial(
        pl.kernel,
        mesh=mesh,
        out_type=jax.ShapeDtypeStruct((batch, hist, DIM), jnp.float32),
        scratch_types=[
            pltpu.VMEM((rows_per_chunk,), jnp.int32),
            pltpu.VMEM((rows_per_chunk,), jnp.int32),
            pltpu.VMEM((BB, hist, DIM), jnp.float32),
            pltpu.VMEM((BB, hist, DIM), jnp.float32),
            pltpu.SemaphoreType.DMA,
            pltpu.SemaphoreType.DMA,
            pltpu.SemaphoreType.DMA,
            pltpu.SemaphoreType.DMA,
            pltpu.SemaphoreType.DMA,
            pltpu.SemaphoreType.DMA,
        ],
        compiler_params=pltpu.CompilerParams(use_tc_tiling_on_sc=False),
    )
    def gather_kernel(table_hbm, idx_hbm, out_hbm,
                      idx0, idx1, rows0, rows1,
                      si0, si1, sg0, sg1, so0, so1):
        wid = lax.axis_index("s") * NC + lax.axis_index("c")
        base = wid * n_chunks
        idx_v = (idx0, idx1)
        rows_v = (rows0, rows1)
        si = (si0, si1)
        sg = (sg0, sg1)
        so = (so0, so1)

        pltpu.async_copy(idx_hbm.at[base], idx0, si0)
        pltpu.async_copy(idx_hbm.at[base + 1], idx1, si1)

        def step(par, g):
            c = base + g
            b0 = c * BB
            @pl.when(g >= 2)
            def _():
                pltpu.make_async_copy(
                    rows_v[par], out_hbm.at[pl.ds(b0, BB)], so[par]).wait()
            pltpu.make_async_copy(idx_hbm.at[c], idx_v[par], si[par]).wait()
            copies = [
                pltpu.async_copy(
                    table_hbm.at[idx_v[par].at[pl.ds(j * hist, hist)]],
                    rows_v[par].at[j], sg[par])
                for j in range(BB)
            ]
            for cp in copies:
                cp.wait()
            pltpu.async_copy(rows_v[par], out_hbm.at[pl.ds(b0, BB)], so[par])
            @pl.when(g + 2 < n_chunks)
            def _():
                pltpu.async_copy(idx_hbm.at[c + 2], idx_v[par], si[par])

        def body(i, carry):
            step(0, 2 * i)
            step(1, 2 * i + 1)
            return carry

        lax.fori_loop(0, n_chunks // 2, body, 0)

        pltpu.make_async_copy(rows0, out_hbm.at[pl.ds(0, BB)], so0).wait()
        pltpu.make_async_copy(rows1, out_hbm.at[pl.ds(0, BB)], so1).wait()

    return gather_kernel


def kernel(residue_type, weight):
    b, h = residue_type.shape
    idx = residue_type.astype(jnp.int32).reshape(-1, BB * h)
    return _make_gather(b, h)(weight, idx)
